# baseline (speedup 1.0000x reference)
import functools

import jax
import jax.numpy as jnp
from jax import lax
from jax.experimental import pallas as pl
from jax.experimental.pallas import tpu as pltpu
from jax.experimental.pallas import tpu_sc as plsc

D = 64
NC = 2
NS = 16
NW = NC * NS
CP = 1024
BB = 128


def _project_table(tableT, mat):
    V = tableT.shape[1]
    nb = pl.cdiv(V, 2 * CP)

    def body(t_ref, m_ref, o_ref):
        dn = (((0,), (1,)), ((), ()))
        e = lax.dot_general(t_ref[:, :CP], m_ref[...], dn,
                            preferred_element_type=jnp.float32)
        o = lax.dot_general(t_ref[:, CP:], m_ref[...], dn,
                            preferred_element_type=jnp.float32)
        o_ref[...] = jnp.concatenate([e, o], axis=1)

    return pl.pallas_call(
        body,
        grid=(nb,),
        in_specs=[
            pl.BlockSpec((D, 2 * CP), lambda i: (0, i)),
            pl.BlockSpec((D, D), lambda i: (0, 0)),
        ],
        out_specs=pl.BlockSpec((CP, 2 * D), lambda i: (i, 0)),
        out_shape=jax.ShapeDtypeStruct((nb * CP, 2 * D), jnp.float32),
    )(tableT, mat)


def _gather_transposed(p, xgT, batch, length):
    nbb = batch // BB
    mesh = plsc.VectorSubcoreMesh(core_axis_name="c", subcore_axis_name="s")

    @functools.partial(
        pl.kernel,
        mesh=mesh,
        out_type=jax.ShapeDtypeStruct((length, D // 8, nbb, 8 * BB),
                                      jnp.float32),
        scratch_types=[
            pltpu.VMEM((length, BB), jnp.int32),
            pltpu.VMEM((BB, D), jnp.float32),
            pltpu.VMEM((BB, D), jnp.float32),
            pltpu.VMEM((D * BB,), jnp.float32),
            pltpu.VMEM((D * BB,), jnp.float32),
            pltpu.SemaphoreType.DMA,
            pltpu.SemaphoreType.DMA,
            pltpu.SemaphoreType.DMA,
            pltpu.SemaphoreType.DMA,
        ],
        compiler_params=pltpu.CompilerParams(use_tc_tiling_on_sc=False,
                                             needs_layout_passes=False),
    )
    def k(p_hbm, x_hbm, out_hbm, idx_v, g0_v, g1_v, s0_v, s1_v,
          gsem0, gsem1, ssem0, ssem1):
        g_v = (g0_v, g1_v)
        s_v = (s0_v, s1_v)
        gsem = (gsem0, gsem1)
        ssem = (ssem0, ssem1)
        wid = lax.axis_index("s") * NC + lax.axis_index("c")
        b0 = wid * BB
        pltpu.sync_copy(x_hbm.at[:, pl.ds(b0, BB)], idx_v)
        iot = lax.iota(jnp.int32, 16)
        pltpu.async_copy(p_hbm.at[idx_v.at[0]], g_v[0], gsem[0])
        pltpu.async_copy(p_hbm.at[idx_v.at[1]], g_v[1], gsem[1])

        @pl.loop(0, length // 2)
        def body(g):
            for ph in range(2):
                l = 2 * g + ph
                pltpu.make_async_copy(p_hbm.at[idx_v.at[l]],
                                      g_v[ph], gsem[ph]).wait()
                @pl.when(g >= 1)
                def _():
                    for r in range(D // 8):
                        pltpu.make_async_copy(
                            s_v[ph].at[pl.ds(1024 * r, 1024)],
                            out_hbm.at[l - 2, r, wid], ssem[ph]).wait()

                for t in range(BB):
                    for m in range(D // 16):
                        v = g_v[ph][t, pl.ds(16 * m, 16)]
                        sidx = (iot + (16 * m)) * BB + t
                        plsc.store_scatter(s_v[ph], [sidx], v)
                for r in range(D // 8):
                    pltpu.async_copy(s_v[ph].at[pl.ds(1024 * r, 1024)],
                                     out_hbm.at[l, r, wid], ssem[ph])

                @pl.when(l + 2 < length)
                def _():
                        pltpu.async_copy(p_hbm.at[idx_v.at[l + 2]],
                                         g_v[ph], gsem[ph])

        for ph in range(2):
            for r in range(D // 8):
                pltpu.make_async_copy(
                    s_v[ph].at[pl.ds(1024 * r, 1024)],
                    out_hbm.at[length - 2 + ph, r, wid], ssem[ph]).wait()

    return k(p, xgT)


def kernel(x, table, mat):
    batch, length = x.shape
    ppack = _project_table(table.T, mat)
    p = ppack.reshape(-1, D)
    xg = x.astype(jnp.int32)
    xg = (xg & ~jnp.int32(2047)) + ((xg & 1023) << 1) + ((xg >> 10) & 1)
    out4 = _gather_transposed(p, xg.T, batch, length)
    out5 = out4.reshape(length, D // 8, batch // BB, 8, BB)
    t1 = out5.transpose(2, 4, 0, 1, 3)
    return t1.reshape(batch, length, D)


# speedup vs baseline: 1.3246x; 1.3246x over previous
import functools

import jax
import jax.numpy as jnp
from jax import lax
from jax.experimental import pallas as pl
from jax.experimental.pallas import tpu as pltpu
from jax.experimental.pallas import tpu_sc as plsc

D = 64
NC = 2
NS = 16
NW = NC * NS
RB = 8
CP = 1024


def _project_table(tableT, mat):
    V = tableT.shape[1]
    nb = pl.cdiv(V, 2 * CP)

    def body(t_ref, m_ref, o_ref):
        dn = (((0,), (1,)), ((), ()))
        e = lax.dot_general(t_ref[:, :CP], m_ref[...], dn,
                            preferred_element_type=jnp.float32)
        o = lax.dot_general(t_ref[:, CP:], m_ref[...], dn,
                            preferred_element_type=jnp.float32)
        o_ref[...] = jnp.concatenate([e, o], axis=1)

    return pl.pallas_call(
        body,
        grid=(nb,),
        in_specs=[
            pl.BlockSpec((D, 2 * CP), lambda i: (0, i)),
            pl.BlockSpec((D, D), lambda i: (0, 0)),
        ],
        out_specs=pl.BlockSpec((CP, 2 * D), lambda i: (i, 0)),
        out_shape=jax.ShapeDtypeStruct((nb * CP, 2 * D), jnp.float32),
    )(tableT, mat)


def _gather_rows(p, x):
    B, L = x.shape
    rows_per_w = B // NW
    mesh = plsc.VectorSubcoreMesh(core_axis_name="c", subcore_axis_name="s")

    @functools.partial(
        pl.kernel,
        mesh=mesh,
        out_type=jax.ShapeDtypeStruct((B, L, D), jnp.float32),
        scratch_types=[
            pltpu.VMEM((RB, L), jnp.int32),
            pltpu.VMEM((RB, L, D), jnp.float32),
            pltpu.SemaphoreType.DMA,
            pltpu.SemaphoreType.DMA,
        ],
        compiler_params=pltpu.CompilerParams(use_tc_tiling_on_sc=False),
    )
    def k(p_hbm, x_hbm, out_hbm, idx_v, rows_v, gsem, osem):
        wid = lax.axis_index("s") * NC + lax.axis_index("c")
        row0 = wid * rows_per_w

        @pl.loop(0, rows_per_w // RB)
        def group(g):
            base = row0 + g * RB
            pltpu.sync_copy(x_hbm.at[pl.ds(base, RB)], idx_v)
            gathers = []
            for b in range(RB):
                gathers.append(pltpu.async_copy(
                    p_hbm.at[idx_v.at[b, pl.ds(0, 128)]],
                    rows_v.at[b, pl.ds(0, 128)], gsem))
                gathers.append(pltpu.async_copy(
                    p_hbm.at[idx_v.at[b, pl.ds(128, L - 128)]],
                    rows_v.at[b, pl.ds(128, L - 128)], gsem))
            stores = []
            for b in range(RB):
                gathers[2 * b].wait()
                gathers[2 * b + 1].wait()
                stores.append(
                    pltpu.async_copy(rows_v.at[b], out_hbm.at[base + b], osem))
            for s in stores:
                s.wait()

    return k(p, x)


def kernel(x, table, mat):
    ppack = _project_table(table.T, mat)
    p = ppack.reshape(-1, D)
    xg = x.astype(jnp.int32)
    xg = (xg & ~jnp.int32(2047)) + ((xg & 1023) << 1) + ((xg >> 10) & 1)
    return _gather_rows(p, xg)


# speedup vs baseline: 1.5050x; 1.1362x over previous
import functools

import jax
import jax.numpy as jnp
from jax import lax
from jax.experimental import pallas as pl
from jax.experimental.pallas import tpu as pltpu
from jax.experimental.pallas import tpu_sc as plsc

D = 64
NC = 2
NS = 16
NW = NC * NS
RB = 8
CP = 2048


def _project_table(tableT, mat):
    V = tableT.shape[1]
    nb = pl.cdiv(V, 2 * CP)

    def body(t_ref, m_ref, o_ref):
        dn = (((0,), (1,)), ((), ()))
        e = lax.dot_general(t_ref[:, :CP], m_ref[...], dn,
                            preferred_element_type=jnp.float32)
        o = lax.dot_general(t_ref[:, CP:], m_ref[...], dn,
                            preferred_element_type=jnp.float32)
        o_ref[...] = jnp.concatenate([e, o], axis=1)

    return pl.pallas_call(
        body,
        grid=(nb,),
        in_specs=[
            pl.BlockSpec((D, 2 * CP), lambda i: (0, i)),
            pl.BlockSpec((D, D), lambda i: (0, 0)),
        ],
        out_specs=pl.BlockSpec((CP, 2 * D), lambda i: (i, 0)),
        out_shape=jax.ShapeDtypeStruct((nb * CP, 2 * D), jnp.float32),
    )(tableT, mat)


def _gather_rows(p, x):
    B, L = x.shape
    rows_per_w = B // NW
    mesh = plsc.VectorSubcoreMesh(core_axis_name="c", subcore_axis_name="s")

    @functools.partial(
        pl.kernel,
        mesh=mesh,
        out_type=jax.ShapeDtypeStruct((B, L, D), jnp.float32),
        scratch_types=[
            pltpu.VMEM((RB, L), jnp.int32),
            pltpu.VMEM((RB, L, D), jnp.float32),
            pltpu.SemaphoreType.DMA,
            pltpu.SemaphoreType.DMA,
        ],
        compiler_params=pltpu.CompilerParams(use_tc_tiling_on_sc=False),
    )
    def k(p_hbm, x_hbm, out_hbm, idx_v, rows_v, gsem, osem):
        wid = lax.axis_index("s") * NC + lax.axis_index("c")
        row0 = wid * rows_per_w

        @pl.loop(0, rows_per_w // RB)
        def group(g):
            base = row0 + g * RB
            pltpu.sync_copy(x_hbm.at[pl.ds(base, RB)], idx_v)
            gathers = []
            for b in range(RB):
                gathers.append(pltpu.async_copy(
                    p_hbm.at[idx_v.at[b, pl.ds(0, 128)]],
                    rows_v.at[b, pl.ds(0, 128)], gsem))
                gathers.append(pltpu.async_copy(
                    p_hbm.at[idx_v.at[b, pl.ds(128, L - 128)]],
                    rows_v.at[b, pl.ds(128, L - 128)], gsem))
            stores = []
            for b in range(RB):
                gathers[2 * b].wait()
                gathers[2 * b + 1].wait()
                stores.append(
                    pltpu.async_copy(rows_v.at[b], out_hbm.at[base + b], osem))
            for s in stores:
                s.wait()

    return k(p, x)


def kernel(x, table, mat):
    ppack = _project_table(table.T, mat)
    p = ppack.reshape(-1, D)
    xg = x.astype(jnp.int32)
    xg = (xg & ~jnp.int32(4095)) + ((xg & 2047) << 1) + ((xg >> 11) & 1)
    return _gather_rows(p, xg)


# speedup vs baseline: 1.6159x; 1.0737x over previous
import functools

import jax
import jax.numpy as jnp
from jax import lax
from jax.experimental import pallas as pl
from jax.experimental.pallas import tpu as pltpu
from jax.experimental.pallas import tpu_sc as plsc

D = 64
NC = 2
NS = 16
NW = NC * NS
RB = 8
CP = 4096


def _project_table(tableT, mat):
    V = tableT.shape[1]
    nb = pl.cdiv(V, 2 * CP)

    def body(t_ref, m_ref, o_ref):
        dn = (((0,), (1,)), ((), ()))
        e = lax.dot_general(t_ref[:, :CP], m_ref[...], dn,
                            preferred_element_type=jnp.float32)
        o = lax.dot_general(t_ref[:, CP:], m_ref[...], dn,
                            preferred_element_type=jnp.float32)
        o_ref[...] = jnp.concatenate([e, o], axis=1)

    return pl.pallas_call(
        body,
        grid=(nb,),
        in_specs=[
            pl.BlockSpec((D, 2 * CP), lambda i: (0, i)),
            pl.BlockSpec((D, D), lambda i: (0, 0)),
        ],
        out_specs=pl.BlockSpec((CP, 2 * D), lambda i: (i, 0)),
        out_shape=jax.ShapeDtypeStruct((nb * CP, 2 * D), jnp.float32),
    )(tableT, mat)


def _gather_rows(p, x):
    B, L = x.shape
    rows_per_w = B // NW
    mesh = plsc.VectorSubcoreMesh(core_axis_name="c", subcore_axis_name="s")

    @functools.partial(
        pl.kernel,
        mesh=mesh,
        out_type=jax.ShapeDtypeStruct((B, L, D), jnp.float32),
        scratch_types=[
            pltpu.VMEM((RB, L), jnp.int32),
            pltpu.VMEM((RB, L, D), jnp.float32),
            pltpu.SemaphoreType.DMA,
            pltpu.SemaphoreType.DMA,
        ],
        compiler_params=pltpu.CompilerParams(use_tc_tiling_on_sc=False),
    )
    def k(p_hbm, x_hbm, out_hbm, idx_v, rows_v, gsem, osem):
        wid = lax.axis_index("s") * NC + lax.axis_index("c")
        row0 = wid * rows_per_w

        @pl.loop(0, rows_per_w // RB)
        def group(g):
            base = row0 + g * RB
            pltpu.sync_copy(x_hbm.at[pl.ds(base, RB)], idx_v)
            gathers = []
            for b in range(RB):
                gathers.append(pltpu.async_copy(
                    p_hbm.at[idx_v.at[b, pl.ds(0, 128)]],
                    rows_v.at[b, pl.ds(0, 128)], gsem))
                gathers.append(pltpu.async_copy(
                    p_hbm.at[idx_v.at[b, pl.ds(128, L - 128)]],
                    rows_v.at[b, pl.ds(128, L - 128)], gsem))
            stores = []
            for b in range(RB):
                gathers[2 * b].wait()
                gathers[2 * b + 1].wait()
                stores.append(
                    pltpu.async_copy(rows_v.at[b], out_hbm.at[base + b], osem))
            for s in stores:
                s.wait()

    return k(p, x)


def kernel(x, table, mat):
    ppack = _project_table(table.T, mat)
    p = ppack.reshape(-1, D)
    xg = x.astype(jnp.int32)
    xg = (xg & ~jnp.int32(8191)) + ((xg & 4095) << 1) + ((xg >> 12) & 1)
    return _gather_rows(p, xg)


# speedup vs baseline: 1.6762x; 1.0373x over previous
import functools

import jax
import jax.numpy as jnp
from jax import lax
from jax.experimental import pallas as pl
from jax.experimental.pallas import tpu as pltpu
from jax.experimental.pallas import tpu_sc as plsc

D = 64
NC = 2
NS = 16
NW = NC * NS
RB = 8
CP = 8192


def _project_table(tableT, mat):
    V = tableT.shape[1]
    nb = pl.cdiv(V, 2 * CP)

    def body(t_ref, m_ref, o_ref):
        dn = (((0,), (1,)), ((), ()))
        e = lax.dot_general(t_ref[:, :CP], m_ref[...], dn,
                            preferred_element_type=jnp.float32)
        o = lax.dot_general(t_ref[:, CP:], m_ref[...], dn,
                            preferred_element_type=jnp.float32)
        o_ref[...] = jnp.concatenate([e, o], axis=1)

    return pl.pallas_call(
        body,
        grid=(nb,),
        in_specs=[
            pl.BlockSpec((D, 2 * CP), lambda i: (0, i)),
            pl.BlockSpec((D, D), lambda i: (0, 0)),
        ],
        out_specs=pl.BlockSpec((CP, 2 * D), lambda i: (i, 0)),
        out_shape=jax.ShapeDtypeStruct((nb * CP, 2 * D), jnp.float32),
    )(tableT, mat)


def _gather_rows(p, x):
    B, L = x.shape
    rows_per_w = B // NW
    mesh = plsc.VectorSubcoreMesh(core_axis_name="c", subcore_axis_name="s")

    @functools.partial(
        pl.kernel,
        mesh=mesh,
        out_type=jax.ShapeDtypeStruct((B, L, D), jnp.float32),
        scratch_types=[
            pltpu.VMEM((RB, L), jnp.int32),
            pltpu.VMEM((RB, L, D), jnp.float32),
            pltpu.SemaphoreType.DMA,
            pltpu.SemaphoreType.DMA,
        ],
        compiler_params=pltpu.CompilerParams(use_tc_tiling_on_sc=False),
    )
    def k(p_hbm, x_hbm, out_hbm, idx_v, rows_v, gsem, osem):
        wid = lax.axis_index("s") * NC + lax.axis_index("c")
        row0 = wid * rows_per_w

        @pl.loop(0, rows_per_w // RB)
        def group(g):
            base = row0 + g * RB
            pltpu.sync_copy(x_hbm.at[pl.ds(base, RB)], idx_v)
            gathers = []
            for b in range(RB):
                gathers.append(pltpu.async_copy(
                    p_hbm.at[idx_v.at[b, pl.ds(0, 128)]],
                    rows_v.at[b, pl.ds(0, 128)], gsem))
                gathers.append(pltpu.async_copy(
                    p_hbm.at[idx_v.at[b, pl.ds(128, L - 128)]],
                    rows_v.at[b, pl.ds(128, L - 128)], gsem))
            stores = []
            for b in range(RB):
                gathers[2 * b].wait()
                gathers[2 * b + 1].wait()
                stores.append(
                    pltpu.async_copy(rows_v.at[b], out_hbm.at[base + b], osem))
            for s in stores:
                s.wait()

    return k(p, x)


def kernel(x, table, mat):
    ppack = _project_table(table.T, mat)
    p = ppack.reshape(-1, D)
    xg = x.astype(jnp.int32)
    xg = (xg & ~jnp.int32(16383)) + ((xg & 8191) << 1) + ((xg >> 13) & 1)
    return _gather_rows(p, xg)


# speedup vs baseline: 1.6995x; 1.0140x over previous
import functools

import jax
import jax.numpy as jnp
from jax import lax
from jax.experimental import pallas as pl
from jax.experimental.pallas import tpu as pltpu
from jax.experimental.pallas import tpu_sc as plsc

D = 64
NC = 2
NS = 16
NW = NC * NS
RB = 8
CP = 16384


def _project_table(tableT, mat):
    V = tableT.shape[1]
    nb = pl.cdiv(V, 2 * CP)

    def body(t_ref, m_ref, o_ref):
        dn = (((0,), (1,)), ((), ()))
        e = lax.dot_general(t_ref[:, :CP], m_ref[...], dn,
                            preferred_element_type=jnp.float32)
        o = lax.dot_general(t_ref[:, CP:], m_ref[...], dn,
                            preferred_element_type=jnp.float32)
        o_ref[...] = jnp.concatenate([e, o], axis=1)

    return pl.pallas_call(
        body,
        grid=(nb,),
        in_specs=[
            pl.BlockSpec((D, 2 * CP), lambda i: (0, i)),
            pl.BlockSpec((D, D), lambda i: (0, 0)),
        ],
        out_specs=pl.BlockSpec((CP, 2 * D), lambda i: (i, 0)),
        out_shape=jax.ShapeDtypeStruct((nb * CP, 2 * D), jnp.float32),
    )(tableT, mat)


def _gather_rows(p, x):
    B, L = x.shape
    rows_per_w = B // NW
    mesh = plsc.VectorSubcoreMesh(core_axis_name="c", subcore_axis_name="s")

    @functools.partial(
        pl.kernel,
        mesh=mesh,
        out_type=jax.ShapeDtypeStruct((B, L, D), jnp.float32),
        scratch_types=[
            pltpu.VMEM((RB, L), jnp.int32),
            pltpu.VMEM((RB, L, D), jnp.float32),
            pltpu.SemaphoreType.DMA,
            pltpu.SemaphoreType.DMA,
        ],
        compiler_params=pltpu.CompilerParams(use_tc_tiling_on_sc=False),
    )
    def k(p_hbm, x_hbm, out_hbm, idx_v, rows_v, gsem, osem):
        wid = lax.axis_index("s") * NC + lax.axis_index("c")
        row0 = wid * rows_per_w

        @pl.loop(0, rows_per_w // RB)
        def group(g):
            base = row0 + g * RB
            pltpu.sync_copy(x_hbm.at[pl.ds(base, RB)], idx_v)
            gathers = []
            for b in range(RB):
                gathers.append(pltpu.async_copy(
                    p_hbm.at[idx_v.at[b, pl.ds(0, 128)]],
                    rows_v.at[b, pl.ds(0, 128)], gsem))
                gathers.append(pltpu.async_copy(
                    p_hbm.at[idx_v.at[b, pl.ds(128, L - 128)]],
                    rows_v.at[b, pl.ds(128, L - 128)], gsem))
            stores = []
            for b in range(RB):
                gathers[2 * b].wait()
                gathers[2 * b + 1].wait()
                stores.append(
                    pltpu.async_copy(rows_v.at[b], out_hbm.at[base + b], osem))
            for s in stores:
                s.wait()

    return k(p, x)


def kernel(x, table, mat):
    ppack = _project_table(table.T, mat)
    p = ppack.reshape(-1, D)
    xg = x.astype(jnp.int32)
    xg = (xg & ~jnp.int32(32767)) + ((xg & 16383) << 1) + ((xg >> 14) & 1)
    return _gather_rows(p, xg)
